# Initial kernel scaffold; baseline (speedup 1.0000x reference)
import jax
import jax.numpy as jnp
from jax.experimental import pallas as pl


def kernel(predicted_offsets, predicted_classes, targets, default_boxes):
    raise NotImplementedError("write your pallas kernel here")


# speedup vs baseline: 154.0123x; 154.0123x over previous
import functools

import jax
import jax.numpy as jnp
from jax import lax
from jax.experimental import pallas as pl
from jax.experimental.pallas import tpu as pltpu

_NUM_CLASSES = 21
_D = 8732
_BATCH = 8
_A = 20
_MATCH_THRESH = 0.5

_DP = 8832
_CP = 24


def _loss_kernel(ann_ref, db_ref, pc_ref, po_ref, out_ref):
    f32 = jnp.float32
    d_iota = lax.broadcasted_iota(jnp.int32, (1, _DP), 1).astype(f32)

    dcx = db_ref[0:1, :]
    dcy = db_ref[1:2, :]
    dw = db_ref[2:3, :]
    dh = db_ref[3:4, :]
    dlx = jnp.maximum(dcx - dw * 0.5, 0.0)
    dly = jnp.maximum(dcy - dh * 0.5, 0.0)
    drx = jnp.minimum(dcx + dw * 0.5, 1.0)
    dry = jnp.minimum(dcy + dh * 0.5, 1.0)
    darea = (drx - dlx) * (dry - dly)

    best = jnp.full((1, _DP), -1.0, f32)
    bwa = jnp.zeros((1, _DP), f32)
    awb = []
    iou_rows = []
    for a in range(_A):
        acx = ann_ref[a, 2]
        acy = ann_ref[a, 3]
        aw = ann_ref[a, 4]
        ah = ann_ref[a, 5]
        alx = jnp.maximum(acx - aw * 0.5, 0.0)
        aly = jnp.maximum(acy - ah * 0.5, 0.0)
        arx = jnp.minimum(acx + aw * 0.5, 1.0)
        ary = jnp.minimum(acy + ah * 0.5, 1.0)
        aarea = (arx - alx) * (ary - aly)
        ix = jnp.maximum(jnp.minimum(drx, arx) - jnp.maximum(dlx, alx), 0.0)
        iy = jnp.maximum(jnp.minimum(dry, ary) - jnp.maximum(dly, aly), 0.0)
        inter = ix * iy
        iou = inter / (darea + aarea - inter + 1e-10)
        iou_rows.append(iou)
        upd = iou > best
        bwa = jnp.where(upd, f32(a), bwa)
        best = jnp.where(upd, iou, best)
        mval = jnp.max(iou)
        awb.append(jnp.min(jnp.where(iou == mval, d_iota, f32(1e9))))

    matched = best >= _MATCH_THRESH
    for a in range(_A):
        hit = d_iota == awb[a]
        matched = jnp.logical_or(matched, hit)
        bwa = jnp.where(hit, f32(a), bwa)

    matched_f = matched.astype(f32)
    n_pos = jnp.sum(matched_f)

    tcx = jnp.zeros((1, _DP), f32)
    tcy = jnp.zeros((1, _DP), f32)
    tw = jnp.zeros((1, _DP), f32)
    th = jnp.zeros((1, _DP), f32)
    tcls = jnp.zeros((1, _DP), f32)
    for a in range(_A):
        sel = bwa == f32(a)
        tcx = jnp.where(sel, ann_ref[a, 2], tcx)
        tcy = jnp.where(sel, ann_ref[a, 3], tcy)
        tw = jnp.where(sel, ann_ref[a, 4], tw)
        th = jnp.where(sel, ann_ref[a, 5], th)
        tcls = jnp.where(sel, ann_ref[a, 1], tcls)

    safe_w = jnp.where(dw > 0.0, dw, 1.0)
    safe_h = jnp.where(dh > 0.0, dh, 1.0)
    off0 = (tcx - dcx) / (safe_w * 0.1)
    off1 = (tcy - dcy) / (safe_h * 0.1)
    off2 = jnp.log(jnp.where(tw > 0.0, tw, 1.0) / safe_w) * 5.0
    off3 = jnp.log(jnp.where(th > 0.0, th, 1.0) / safe_h) * 5.0
    offs = (off0, off1, off2, off3)

    ccol = jnp.where(matched, tcls - 1.0, f32(-1.0))

    c_iota = lax.broadcasted_iota(jnp.int32, (_CP, _DP), 0).astype(f32)
    pos_sum = f32(0.0)
    reg_sum = f32(0.0)
    for j in range(_BATCH):
        x = pc_ref[j]
        m = jnp.max(x, axis=0, keepdims=True)
        e = jnp.exp(x - m)
        z = jnp.sum(e, axis=0, keepdims=True)
        e_true = jnp.sum(jnp.where(c_iota == ccol, e, 0.0), axis=0,
                         keepdims=True)
        p = e_true / z
        p = jnp.clip(p, 1e-07, 1.0 - 1e-07)
        fl = -0.25 * jnp.log(p) * (1.0 - p) * (1.0 - p)
        pos_sum = pos_sum + jnp.sum(fl * matched_f)
        for k in range(4):
            d = po_ref[j, k:k + 1, :] - offs[k]
            ad = jnp.abs(d)
            sl1 = jnp.where(ad < 1.0, 0.5 * d * d, ad - 0.5)
            reg_sum = reg_sum + jnp.sum(sl1 * matched_f)

    inv = 1.0 / (f32(_BATCH) * n_pos)
    cls_loss = pos_sum * inv
    reg_loss = reg_sum * inv
    r_iota = lax.broadcasted_iota(jnp.int32, (8, 128), 0)
    l_iota = lax.broadcasted_iota(jnp.int32, (8, 128), 1)
    vals = jnp.where(l_iota == 0, cls_loss + reg_loss,
                     jnp.where(l_iota == 1, reg_loss,
                               jnp.where(l_iota == 2, cls_loss, 0.0)))
    out_ref[:, :] = jnp.where(r_iota == 0, vals, 0.0)


@jax.jit
def kernel(predicted_offsets, predicted_classes, targets, default_boxes):
    f32 = jnp.float32
    ann = jnp.pad(targets[:_A], ((0, 0), (0, 2))).astype(f32)

    db_t = jnp.pad(default_boxes.T, ((0, 4), (0, _DP - _D)))

    pc_t = jnp.transpose(predicted_classes, (0, 2, 1))
    pc_t = jnp.pad(pc_t, ((0, 0), (0, _CP - _NUM_CLASSES), (0, _DP - _D)),
                   constant_values=-1e30)

    po_t = jnp.transpose(predicted_offsets, (0, 2, 1))
    po_t = jnp.pad(po_t, ((0, 0), (0, 4), (0, _DP - _D)))

    out = pl.pallas_call(
        _loss_kernel,
        out_shape=jax.ShapeDtypeStruct((8, 128), f32),
        in_specs=[
            pl.BlockSpec(memory_space=pltpu.SMEM),
            pl.BlockSpec(memory_space=pltpu.VMEM),
            pl.BlockSpec(memory_space=pltpu.VMEM),
            pl.BlockSpec(memory_space=pltpu.VMEM),
        ],
        out_specs=pl.BlockSpec(memory_space=pltpu.VMEM),
    )(ann, db_t, pc_t, po_t)

    total = out[0, 0]
    reg_loss = out[0, 1]
    cls_loss = out[0, 2]
    return (total, reg_loss, cls_loss)


# speedup vs baseline: 160.5625x; 1.0425x over previous
import jax
import jax.numpy as jnp
from jax import lax
from jax.experimental import pallas as pl
from jax.experimental.pallas import tpu as pltpu

_NUM_CLASSES = 21
_D = 8732
_BATCH = 8
_A = 20
_MATCH_THRESH = 0.5

_DP = 8832
_R, _C = 8, 1104


def _loss_kernel(ann_ref, db_ref, pc_ref, po_ref, out_ref):
    f32 = jnp.float32
    shp = (_R, _C)
    d_iota = (lax.broadcasted_iota(jnp.int32, shp, 0) * _C
              + lax.broadcasted_iota(jnp.int32, shp, 1)).astype(f32)

    dcx = db_ref[0]
    dcy = db_ref[1]
    dw = db_ref[2]
    dh = db_ref[3]
    dlx = jnp.maximum(dcx - dw * 0.5, 0.0)
    dly = jnp.maximum(dcy - dh * 0.5, 0.0)
    drx = jnp.minimum(dcx + dw * 0.5, 1.0)
    dry = jnp.minimum(dcy + dh * 0.5, 1.0)
    darea = (drx - dlx) * (dry - dly)

    best = jnp.full(shp, -1.0, f32)
    bwa = jnp.zeros(shp, f32)
    awb = []
    for a in range(_A):
        acx = ann_ref[a, 2]
        acy = ann_ref[a, 3]
        aw = ann_ref[a, 4]
        ah = ann_ref[a, 5]
        alx = jnp.maximum(acx - aw * 0.5, 0.0)
        aly = jnp.maximum(acy - ah * 0.5, 0.0)
        arx = jnp.minimum(acx + aw * 0.5, 1.0)
        ary = jnp.minimum(acy + ah * 0.5, 1.0)
        aarea = (arx - alx) * (ary - aly)
        ix = jnp.maximum(jnp.minimum(drx, arx) - jnp.maximum(dlx, alx), 0.0)
        iy = jnp.maximum(jnp.minimum(dry, ary) - jnp.maximum(dly, aly), 0.0)
        inter = ix * iy
        iou = inter / (darea + aarea - inter + 1e-10)
        upd = iou > best
        bwa = jnp.where(upd, f32(a), bwa)
        best = jnp.where(upd, iou, best)
        mval = jnp.max(iou)
        awb.append(jnp.min(jnp.where(iou == mval, d_iota, f32(1e9))))

    matched = best >= _MATCH_THRESH
    for a in range(_A):
        hit = d_iota == awb[a]
        matched = jnp.logical_or(matched, hit)
        bwa = jnp.where(hit, f32(a), bwa)

    matched_f = matched.astype(f32)
    n_pos = jnp.sum(matched_f)

    tcx = jnp.zeros(shp, f32)
    tcy = jnp.zeros(shp, f32)
    tw = jnp.zeros(shp, f32)
    th = jnp.zeros(shp, f32)
    tcls = jnp.zeros(shp, f32)
    for a in range(_A):
        sel = bwa == f32(a)
        tcx = jnp.where(sel, ann_ref[a, 2], tcx)
        tcy = jnp.where(sel, ann_ref[a, 3], tcy)
        tw = jnp.where(sel, ann_ref[a, 4], tw)
        th = jnp.where(sel, ann_ref[a, 5], th)
        tcls = jnp.where(sel, ann_ref[a, 1], tcls)

    safe_w = jnp.where(dw > 0.0, dw, 1.0)
    safe_h = jnp.where(dh > 0.0, dh, 1.0)
    off0 = (tcx - dcx) / (safe_w * 0.1)
    off1 = (tcy - dcy) / (safe_h * 0.1)
    off2 = jnp.log(jnp.where(tw > 0.0, tw, 1.0) / safe_w) * 5.0
    off3 = jnp.log(jnp.where(th > 0.0, th, 1.0) / safe_h) * 5.0
    offs = (off0, off1, off2, off3)

    ccol = jnp.where(matched, tcls - 1.0, f32(-1.0))

    pos_sum = f32(0.0)
    reg_sum = f32(0.0)
    for j in range(_BATCH):
        rows = [pc_ref[j, c] for c in range(_NUM_CLASSES)]
        m = rows[0]
        for c in range(1, _NUM_CLASSES):
            m = jnp.maximum(m, rows[c])
        z = jnp.zeros(shp, f32)
        e_true = jnp.zeros(shp, f32)
        for c in range(_NUM_CLASSES):
            e = jnp.exp(rows[c] - m)
            z = z + e
            e_true = jnp.where(ccol == f32(c), e, e_true)
        p = e_true / z
        p = jnp.clip(p, 1e-07, 1.0 - 1e-07)
        fl = -0.25 * jnp.log(p) * (1.0 - p) * (1.0 - p)
        pos_sum = pos_sum + jnp.sum(fl * matched_f)
        for k in range(4):
            d = po_ref[j, k] - offs[k]
            ad = jnp.abs(d)
            sl1 = jnp.where(ad < 1.0, 0.5 * d * d, ad - 0.5)
            reg_sum = reg_sum + jnp.sum(sl1 * matched_f)

    inv = 1.0 / (f32(_BATCH) * n_pos)
    cls_loss = pos_sum * inv
    reg_loss = reg_sum * inv
    r_iota = lax.broadcasted_iota(jnp.int32, (8, 128), 0)
    l_iota = lax.broadcasted_iota(jnp.int32, (8, 128), 1)
    vals = jnp.where(l_iota == 0, cls_loss + reg_loss,
                     jnp.where(l_iota == 1, reg_loss,
                               jnp.where(l_iota == 2, cls_loss, 0.0)))
    out_ref[:, :] = jnp.where(r_iota == 0, vals, 0.0)


@jax.jit
def kernel(predicted_offsets, predicted_classes, targets, default_boxes):
    f32 = jnp.float32
    ann = jnp.pad(targets[:_A], ((0, 0), (0, 2))).astype(f32)

    db_t = jnp.pad(default_boxes, ((0, _DP - _D), (0, 0)))
    db_t = jnp.transpose(db_t.reshape(_R, _C, 4), (2, 0, 1))

    pc_t = jnp.pad(predicted_classes, ((0, 0), (0, _DP - _D), (0, 0)))
    pc_t = jnp.transpose(pc_t.reshape(_BATCH, _R, _C, _NUM_CLASSES),
                         (0, 3, 1, 2))

    po_t = jnp.pad(predicted_offsets, ((0, 0), (0, _DP - _D), (0, 0)))
    po_t = jnp.transpose(po_t.reshape(_BATCH, _R, _C, 4),
                         (0, 3, 1, 2))

    out = pl.pallas_call(
        _loss_kernel,
        out_shape=jax.ShapeDtypeStruct((8, 128), f32),
        in_specs=[
            pl.BlockSpec(memory_space=pltpu.SMEM),
            pl.BlockSpec(memory_space=pltpu.VMEM),
            pl.BlockSpec(memory_space=pltpu.VMEM),
            pl.BlockSpec(memory_space=pltpu.VMEM),
        ],
        out_specs=pl.BlockSpec(memory_space=pltpu.VMEM),
    )(ann, db_t, pc_t, po_t)

    total = out[0, 0]
    reg_loss = out[0, 1]
    cls_loss = out[0, 2]
    return (total, reg_loss, cls_loss)
